# baseline (speedup 1.0000x reference)
import functools
import jax
import jax.numpy as jnp
from jax import lax
from jax.experimental import pallas as pl
from jax.experimental.pallas import tpu as pltpu
from jax.experimental.pallas import tpu_sc as plsc

N = 10000
E = 160000
D_HID = 128
D_IN = 13
D_OUT = 64
N_ITERS = 6

NW = 32
NPS = 320
NPAD = NW * NPS
CHUNK = 128
SCHUNK = 1600
NCH = E // SCHUNK
NST = 4
FLUSH = 4096
STAGE = 2 * FLUSH
CAP = 40 * FLUSH
FCAP = CAP // NST
HB = 384
WIN = 16384
HCH = 2048
ROW_BLK = 1000


def _pre_body(ni_ref, wcat_ref, bcat_ref, wh_ref, h0_ref, g_ref, q0_ref):
    t = jnp.dot(ni_ref[...], wcat_ref[...], preferred_element_type=jnp.float32)
    t = t + bcat_ref[...]
    h0 = jnp.maximum(t[:, :D_HID], 0.0)
    g = t[:, D_HID:]
    h0_ref[...] = h0
    g_ref[...] = g
    q0_ref[...] = jnp.dot(h0, wh_ref[...], preferred_element_type=jnp.float32) + g


def _upd_body(m_ref, h_ref, g_ref, wu1_ref, wu2_ref, bu_ref, wh_ref,
              hn_ref, qn_ref):
    hn = jnp.dot(m_ref[...], wu1_ref[...], preferred_element_type=jnp.float32)
    hn = hn + jnp.dot(h_ref[...], wu2_ref[...], preferred_element_type=jnp.float32)
    hn = jnp.maximum(hn + bu_ref[...], 0.0)
    hn_ref[...] = hn
    qn_ref[...] = jnp.dot(hn, wh_ref[...], preferred_element_type=jnp.float32) + g_ref[...]


def _out_body(h_ref, wo_ref, bo_ref, o_ref):
    o = jnp.dot(h_ref[...], wo_ref[...], preferred_element_type=jnp.float32)
    o = o + bo_ref[...]
    nrm = jnp.sqrt(jnp.sum(o * o, axis=-1, keepdims=True))
    o_ref[...] = o / jnp.maximum(nrm, 1e-12)


def _full(shape):
    return pl.BlockSpec(shape, lambda i: (0, 0))


def _rows(width):
    return pl.BlockSpec((ROW_BLK, width), lambda i: (i, 0))


_GRID = N // ROW_BLK

_pre_call = pl.pallas_call(
    _pre_body,
    grid=(_GRID,),
    in_specs=[_rows(D_IN), _full((D_IN, 2 * D_HID)), _full((1, 2 * D_HID)),
              _full((D_HID, D_HID))],
    out_specs=[_rows(D_HID), _rows(D_HID), _rows(D_HID)],
    out_shape=[jax.ShapeDtypeStruct((N, D_HID), jnp.float32)] * 3,
)

_upd_call = pl.pallas_call(
    _upd_body,
    grid=(_GRID,),
    in_specs=[_rows(D_HID), _rows(D_HID), _rows(D_HID),
              _full((D_HID, D_HID)), _full((D_HID, D_HID)),
              _full((1, D_HID)), _full((D_HID, D_HID))],
    out_specs=[_rows(D_HID), _rows(D_HID)],
    out_shape=[jax.ShapeDtypeStruct((N, D_HID), jnp.float32)] * 2,
)

_out_call = pl.pallas_call(
    _out_body,
    grid=(_GRID,),
    in_specs=[_rows(D_HID), _full((D_HID, D_OUT)), _full((1, D_OUT))],
    out_specs=_rows(D_OUT),
    out_shape=jax.ShapeDtypeStruct((N, D_OUT), jnp.float32),
)


def _part_body(src_hbm, dst_hbm, ef_hbm,
               srcl_hbm, dstl_hbm, efl_hbm, cnt_hbm,
               sbuf, dbuf, ebuf, sst, dst_st, est, cntv):
    wid = lax.axis_index("s") * 2 + lax.axis_index("c")
    lo = wid * NPS
    iota = lax.iota(jnp.int32, 16)

    def _chunk(c, carry):
        vptrs = list(carry[:NST])
        optrs = list(carry[NST:])
        off = c * SCHUNK
        pltpu.sync_copy(src_hbm.at[pl.ds(off, SCHUNK)], sbuf)
        pltpu.sync_copy(dst_hbm.at[pl.ds(off, SCHUNK)], dbuf)
        pltpu.sync_copy(ef_hbm.at[pl.ds(off, SCHUNK)], ebuf)

        def _grp(it, vps):
            out = []
            for st in range(NST):
                gi = it * NST + st
                sl = pl.ds(gi * 16, 16)
                d16 = dbuf[sl]
                m = (d16 >= lo) & (d16 < lo + NPS)
                plsc.store_compressed(sst.at[pl.ds(st * STAGE + vps[st], 16)],
                                      sbuf[sl], mask=m)
                plsc.store_compressed(dst_st.at[pl.ds(st * STAGE + vps[st], 16)],
                                      d16 - lo, mask=m)
                plsc.store_compressed(est.at[pl.ds(st * STAGE + vps[st], 16)],
                                      ebuf[sl], mask=m)
                pc = plsc.all_reduce_population_count(m)[0]
                out.append(vps[st] + pc)
            return tuple(out)
        vptrs = list(lax.fori_loop(0, SCHUNK // (16 * NST), _grp,
                                   tuple(vptrs)))

        for st in range(NST):
            flushed = vptrs[st] >= FLUSH
            obase = pl.multiple_of(wid * CAP + st * FCAP + optrs[st], FLUSH)

            @pl.when(flushed)
            def _(st=st, obase=obase):
                pltpu.sync_copy(sst.at[pl.ds(st * STAGE, FLUSH)],
                                srcl_hbm.at[pl.ds(obase, FLUSH)])
                pltpu.sync_copy(dst_st.at[pl.ds(st * STAGE, FLUSH)],
                                dstl_hbm.at[pl.ds(obase, FLUSH)])
                pltpu.sync_copy(est.at[pl.ds(st * STAGE, FLUSH)],
                                efl_hbm.at[pl.ds(obase, FLUSH)])

                def _mv(i, _):
                    sl_lo = pl.ds(st * STAGE + i * 16, 16)
                    sl_hi = pl.ds(st * STAGE + FLUSH + i * 16, 16)
                    sst[sl_lo] = sst[sl_hi]
                    dst_st[sl_lo] = dst_st[sl_hi]
                    est[sl_lo] = est[sl_hi]
                    return 0
                lax.fori_loop(0, FLUSH // 16, _mv, 0)

            vptrs[st] = jnp.where(flushed, vptrs[st] - FLUSH, vptrs[st])
            optrs[st] = jnp.where(flushed, optrs[st] + FLUSH, optrs[st])
        return tuple(vptrs) + tuple(optrs)

    zero = jnp.int32(0)
    carry = lax.fori_loop(0, NCH, _chunk, (zero,) * (2 * NST))
    vptrs = carry[:NST]
    optrs = carry[NST:]

    cv = jnp.zeros((16,), jnp.int32)
    for st in range(NST):
        for k in range(CHUNK // 16):
            at = pl.ds(st * STAGE + vptrs[st] + k * 16, 16)
            sst[at] = jnp.zeros((16,), jnp.int32)
            dst_st[at] = jnp.full((16,), NPS, jnp.int32)
            est[at] = jnp.zeros((16,), jnp.float32)

        fbase = pl.multiple_of(wid * CAP + st * FCAP + optrs[st], FLUSH)
        pltpu.sync_copy(sst.at[pl.ds(st * STAGE, FLUSH)],
                        srcl_hbm.at[pl.ds(fbase, FLUSH)])
        pltpu.sync_copy(dst_st.at[pl.ds(st * STAGE, FLUSH)],
                        dstl_hbm.at[pl.ds(fbase, FLUSH)])
        pltpu.sync_copy(est.at[pl.ds(st * STAGE, FLUSH)],
                        efl_hbm.at[pl.ds(fbase, FLUSH)])

        total = optrs[st] + ((vptrs[st] + CHUNK - 1) // CHUNK) * CHUNK
        cv = jnp.where(iota == st, total, cv)

    cntv[...] = cv.astype(jnp.float32)
    pltpu.sync_copy(cntv, cnt_hbm.at[pl.ds(wid * 16, 16)])


_part_call = pl.kernel(
    _part_body,
    out_type=[
        jax.ShapeDtypeStruct((NW * CAP,), jnp.int32),
        jax.ShapeDtypeStruct((NW * CAP,), jnp.int32),
        jax.ShapeDtypeStruct((NW * CAP,), jnp.float32),
        jax.ShapeDtypeStruct((NW * 16,), jnp.float32),
    ],
    mesh=plsc.VectorSubcoreMesh(core_axis_name="c", subcore_axis_name="s",
                                num_cores=2, num_subcores=16),
    compiler_params=pltpu.CompilerParams(needs_layout_passes=False),
    scratch_types=[
        pltpu.VMEM((SCHUNK,), jnp.int32),
        pltpu.VMEM((SCHUNK,), jnp.int32),
        pltpu.VMEM((SCHUNK,), jnp.float32),
        pltpu.VMEM((NST * STAGE,), jnp.int32),
        pltpu.VMEM((NST * STAGE,), jnp.int32),
        pltpu.VMEM((NST * STAGE,), jnp.float32),
        pltpu.VMEM((16,), jnp.float32),
    ],
)


def _sort_body(srcl_hbm, dstl_hbm, efl_hbm, cnt_hbm,
               srcs_hbm, dsts_hbm, efs_hbm,
               sbuf, dbuf, ebuf, hist_v, wptr_v,
               s_out, d_out, e_out, cntv):
    wid = lax.axis_index("s") * 2 + lax.axis_index("c")
    base = wid * CAP
    iota = lax.iota(jnp.int32, 16)

    pltpu.sync_copy(cnt_hbm.at[pl.ds(wid * 16, 16)], cntv)
    cvec = cntv[...]
    cnts = [cvec[f].astype(jnp.int32) for f in range(NST)]
    total = cnts[0] + cnts[1] + cnts[2] + cnts[3]

    def _zh(i, _):
        for b in range(NST):
            hist_v[b, pl.ds(i * 16, 16)] = jnp.zeros((16,), jnp.int32)
        return 0
    lax.fori_loop(0, HB // 16, _zh, 0)

    for f in range(NST):
        cnt_f = cnts[f]
        fbase = base + f * FCAP
        nscan = (cnt_f + HCH - 1) // HCH

        def _hchunk(c, _, cnt_f=cnt_f, fbase=fbase):
            off = pl.multiple_of(fbase + c * HCH, 8)
            gbase = c * HCH
            pltpu.sync_copy(dstl_hbm.at[pl.ds(off, HCH)], dbuf)

            def _hgrp(g, _):
                d16 = dbuf[pl.ds(g * 16, 16)]
                valid = (gbase + g * 16 + iota) < cnt_f
                d16c = jnp.where(valid, jnp.clip(d16, 0, NPS), NPS)
                for j2 in range(16):
                    bsp = jnp.full((16,), j2 % 4, jnp.int32)
                    dsp = jnp.full((16,), d16c[j2], jnp.int32)
                    cv = plsc.load_gather(hist_v, [bsp, dsp])
                    plsc.store_scatter(hist_v, [bsp, dsp], cv + 1)
                return 0
            lax.fori_loop(0, HCH // 16, _hgrp, 0)
            return 0
        lax.fori_loop(0, nscan, _hchunk, 0)

    def _prefix():
        carry = jnp.int32(0)
        for g in range(HB // 16):
            sl = pl.ds(g * 16, 16)
            h = [hist_v[b, sl] for b in range(NST)]
            tot16 = h[0] + h[1] + h[2] + h[3]
            vals = jnp.zeros((16,), jnp.int32)
            for j2 in range(16):
                vals = jnp.where(iota == j2, carry, vals)
                carry = carry + tot16[j2]
            wptr_v[0, sl] = vals
            wptr_v[1, sl] = vals + h[0]
            wptr_v[2, sl] = vals + h[0] + h[1]
            wptr_v[3, sl] = vals + h[0] + h[1] + h[2]

    nrounds = (total + WIN - 1) // WIN

    def _round(r, _):
        lo_pos = r * WIN
        _prefix()
        lane0 = iota == 0

        for f in range(NST):
            cnt_f = cnts[f]
            fbase = base + f * FCAP
            nscan = (cnt_f + HCH - 1) // HCH

            def _schunk(c, _, cnt_f=cnt_f, fbase=fbase):
                off = pl.multiple_of(fbase + c * HCH, 8)
                gbase = c * HCH
                pltpu.sync_copy(srcl_hbm.at[pl.ds(off, HCH)], sbuf)
                pltpu.sync_copy(dstl_hbm.at[pl.ds(off, HCH)], dbuf)
                pltpu.sync_copy(efl_hbm.at[pl.ds(off, HCH)], ebuf)

                def _sgrp(g, _):
                    sl = pl.ds(g * 16, 16)
                    valid = (gbase + g * 16 + iota) < cnt_f
                    d16c = jnp.where(valid, jnp.clip(dbuf[sl], 0, NPS), NPS)
                    s16c = jnp.where(valid, sbuf[sl], 0)
                    e16c = jnp.where(valid, ebuf[sl], 0.0)
                    for j2 in range(16):
                        bsp = jnp.full((16,), 0, jnp.int32)
                        dsp = jnp.full((16,), d16c[j2], jnp.int32)
                        pos = plsc.load_gather(wptr_v, [bsp, dsp])
                        plsc.store_scatter(wptr_v, [bsp, dsp], pos + 1)
                        m = (pos >= lo_pos) & (pos < lo_pos + WIN) & lane0
                        pl_loc = pos - lo_pos
                        plsc.store_scatter(
                            s_out, [pl_loc],
                            jnp.full((16,), s16c[j2], jnp.int32), mask=m)
                        plsc.store_scatter(
                            d_out, [pl_loc],
                            jnp.full((16,), d16c[j2], jnp.int32), mask=m)
                        plsc.store_scatter(
                            e_out, [pl_loc],
                            jnp.full((16,), e16c[j2], jnp.float32), mask=m)
                    return 0
                lax.fori_loop(0, HCH // 16, _sgrp, 0)
                return 0
            lax.fori_loop(0, nscan, _schunk, 0)

        wbase = pl.multiple_of(base + r * WIN, 8)
        pltpu.sync_copy(s_out, srcs_hbm.at[pl.ds(wbase, WIN)])
        pltpu.sync_copy(d_out, dsts_hbm.at[pl.ds(wbase, WIN)])
        pltpu.sync_copy(e_out, efs_hbm.at[pl.ds(wbase, WIN)])
        return 0
    lax.fori_loop(0, nrounds, _round, 0)


_sort_call = pl.kernel(
    _sort_body,
    out_type=[
        jax.ShapeDtypeStruct((NW * CAP,), jnp.int32),
        jax.ShapeDtypeStruct((NW * CAP,), jnp.int32),
        jax.ShapeDtypeStruct((NW * CAP,), jnp.float32),
    ],
    mesh=plsc.VectorSubcoreMesh(core_axis_name="c", subcore_axis_name="s",
                                num_cores=2, num_subcores=16),
    compiler_params=pltpu.CompilerParams(needs_layout_passes=False),
    scratch_types=[
        pltpu.VMEM((HCH,), jnp.int32),
        pltpu.VMEM((HCH,), jnp.int32),
        pltpu.VMEM((HCH,), jnp.float32),
        pltpu.VMEM((NST, HB), jnp.int32),
        pltpu.VMEM((NST, HB), jnp.int32),
        pltpu.VMEM((WIN,), jnp.int32),
        pltpu.VMEM((WIN,), jnp.int32),
        pltpu.VMEM((WIN,), jnp.float32),
        pltpu.VMEM((16,), jnp.float32),
    ],
)


def _edge_body(q_hbm, src_hbm, dloc_hbm, ef_hbm, cnt_hbm, wvec_hbm,
               mhat_hbm,
               wv_v, cnt_v, sidx_v, dloc_v, ef_v, rows_v, acc_v, sem):
    wid = lax.axis_index("s") * 2 + lax.axis_index("c")

    pltpu.sync_copy(wvec_hbm, wv_v)
    pltpu.sync_copy(cnt_hbm.at[pl.ds(wid * 16, 16)], cnt_v)

    def _zero(i, _):
        for k in range(8):
            acc_v[i, pl.ds(k * 16, 16)] = jnp.zeros((16,), jnp.float32)
        return 0
    lax.fori_loop(0, NPS + 1, _zero, 0)

    _cv = cnt_v[...]
    nchunks = (_cv[0] + _cv[1] + _cv[2] + _cv[3]).astype(jnp.int32) // CHUNK
    iota = lax.iota(jnp.int32, 16)
    wregs = [wv_v[pl.ds(k * 16, 16)] for k in range(8)]

    def _issue(c, sl):
        off = pl.multiple_of(wid * CAP + c * CHUNK, CHUNK)
        pltpu.sync_copy(src_hbm.at[pl.ds(off, CHUNK)], sidx_v.at[sl])
        pltpu.async_copy(q_hbm.at[sidx_v.at[sl]], rows_v.at[sl], sem.at[sl])
        pltpu.sync_copy(dloc_hbm.at[pl.ds(off, CHUNK)], dloc_v.at[sl])
        pltpu.sync_copy(ef_hbm.at[pl.ds(off, CHUNK)], ef_v.at[sl])

    @pl.when(nchunks > 0)
    def _():
        _issue(0, 0)

    def _chunk(c, carry):
        sl = c % 2

        @pl.when(c + 1 < nchunks)
        def _():
            _issue(c + 1, 1 - sl)

        pltpu.make_async_copy(q_hbm.at[sidx_v.at[sl]], rows_v.at[sl],
                              sem.at[sl]).wait()

        def _grp(gidx, carry):
            prev = carry[0]
            accs = carry[1:]
            d16 = dloc_v[sl, pl.ds(gidx * 16, 16)]
            e16 = ef_v[sl, pl.ds(gidx * 16, 16)]
            for j2 in range(16):
                j = gidx * 16 + j2
                dvec = jnp.full((16,), d16[j2], jnp.int32)
                evec = jnp.full((16,), e16[j2], jnp.float32)
                is_new = dvec != prev
                new_accs = []
                for k in range(8):
                    rk = rows_v[sl, j, pl.ds(k * 16, 16)]
                    ik = iota + (k * 16)
                    vk = rk + evec * wregs[k]
                    plsc.store_scatter(acc_v, [prev, ik], accs[k],
                                       mask=is_new)
                    ak = jnp.where(is_new, vk, jnp.maximum(accs[k], vk))
                    new_accs.append(ak)
                prev = dvec
                accs = new_accs
            return (prev,) + tuple(accs)
        carry = lax.fori_loop(0, CHUNK // 16, _grp, carry)
        return carry

    nps_sp = jnp.full((16,), NPS, jnp.int32)
    zero_v = jnp.zeros((16,), jnp.float32)
    carry0 = (nps_sp,) + (zero_v,) * 8
    carry = lax.fori_loop(0, nchunks, _chunk, carry0)

    prev = carry[0]
    for k in range(8):
        ik = iota + (k * 16)
        plsc.store_scatter(acc_v, [prev, ik], carry[1 + k])

    def _clamp(i, _):
        for k in range(8):
            sle = pl.ds(k * 16, 16)
            acc_v[i, sle] = jnp.maximum(acc_v[i, sle], 0.0)
        return 0
    lax.fori_loop(0, NPS, _clamp, 0)
    pltpu.sync_copy(acc_v.at[pl.ds(0, NPS)], mhat_hbm.at[pl.ds(wid * NPS, NPS)])


_edge_call = pl.kernel(
    _edge_body,
    out_type=jax.ShapeDtypeStruct((NPAD, D_HID), jnp.float32),
    mesh=plsc.VectorSubcoreMesh(core_axis_name="c", subcore_axis_name="s",
                                num_cores=2, num_subcores=16),
    compiler_params=pltpu.CompilerParams(needs_layout_passes=False),
    scratch_types=[
        pltpu.VMEM((D_HID,), jnp.float32),
        pltpu.VMEM((16,), jnp.float32),
        pltpu.VMEM((2, CHUNK), jnp.int32),
        pltpu.VMEM((2, CHUNK), jnp.int32),
        pltpu.VMEM((2, CHUNK), jnp.float32),
        pltpu.VMEM((2, CHUNK, D_HID), jnp.float32),
        pltpu.VMEM((NPS + 1, D_HID), jnp.float32),
        pltpu.SemaphoreType.DMA((2,)),
    ],
)


@jax.jit
def _run(state, edge_index, node_feature, edge_feature,
         W_in, b_in, W_msg, b_msg, W_upd, b_upd, W_out, b_out):
    glob = jnp.broadcast_to(state[0, :5][None, :], (N, 5))
    loc1 = state[0, 5:5 + N][:, None]
    loc2 = state[0, 5 + N:5 + 2 * N][:, None]
    ni = jnp.concatenate([glob, loc1, loc2, node_feature], axis=-1)

    W_h = W_msg[:D_HID]
    w_e = W_msg[D_HID]
    W_s = W_msg[D_HID + 1:]
    Wcat = jnp.concatenate([W_in, W_s], axis=1)
    bcat = jnp.concatenate([b_in, b_msg])[None, :]
    Wu1 = W_upd[:D_HID]
    Wu2 = W_upd[D_HID:]

    src_l, dst_l, ef_l, counts = _part_call(
        edge_index[0], edge_index[1], edge_feature)
    src_l, dst_l, ef_l = _sort_call(src_l, dst_l, ef_l, counts)

    h, g, q = _pre_call(ni, Wcat, bcat, W_h)
    for _ in range(N_ITERS):
        mhat = _edge_call(q, src_l, dst_l, ef_l, counts, w_e)[:N]
        h, q = _upd_call(mhat, h, g, Wu1, Wu2, b_upd[None, :], W_h)
    out = _out_call(h, W_out, b_out[None, :])
    return out[:, None, :]


def kernel(state, edge_index, node_feature, edge_feature,
           W_in, b_in, W_msg, b_msg, W_upd, b_upd, W_out, b_out):
    if state.ndim == 1:
        state = state[None, :]
    return _run(state, edge_index, node_feature, edge_feature,
                W_in, b_in, W_msg, b_msg, W_upd, b_upd, W_out, b_out)


# speedup vs baseline: 1.9025x; 1.9025x over previous
import functools
import jax
import jax.numpy as jnp
from jax import lax
from jax.experimental import pallas as pl
from jax.experimental.pallas import tpu as pltpu
from jax.experimental.pallas import tpu_sc as plsc

N = 10000
E = 160000
D_HID = 128
D_IN = 13
D_OUT = 64
N_ITERS = 6

NW = 32
NPS = 320
NPAD = NW * NPS
CHUNK = 128
SCHUNK = 2000
NCH = E // SCHUNK
FLUSH = 4096
STAGE = 2 * FLUSH
CAP = 40 * FLUSH
WIN = 16384
HCH = 2048
ROW_BLK = 1000


def _pre_body(ni_ref, wcat_ref, bcat_ref, wh_ref, h0_ref, g_ref, q0_ref):
    t = jnp.dot(ni_ref[...], wcat_ref[...], preferred_element_type=jnp.float32)
    t = t + bcat_ref[...]
    h0 = jnp.maximum(t[:, :D_HID], 0.0)
    g = t[:, D_HID:]
    h0_ref[...] = h0
    g_ref[...] = g
    q0_ref[...] = jnp.dot(h0, wh_ref[...], preferred_element_type=jnp.float32) + g


def _upd_body(m_ref, h_ref, g_ref, wu1_ref, wu2_ref, bu_ref, wh_ref,
              hn_ref, qn_ref):
    hn = jnp.dot(m_ref[...], wu1_ref[...], preferred_element_type=jnp.float32)
    hn = hn + jnp.dot(h_ref[...], wu2_ref[...], preferred_element_type=jnp.float32)
    hn = jnp.maximum(hn + bu_ref[...], 0.0)
    hn_ref[...] = hn
    qn_ref[...] = jnp.dot(hn, wh_ref[...], preferred_element_type=jnp.float32) + g_ref[...]


def _out_body(h_ref, wo_ref, bo_ref, o_ref):
    o = jnp.dot(h_ref[...], wo_ref[...], preferred_element_type=jnp.float32)
    o = o + bo_ref[...]
    nrm = jnp.sqrt(jnp.sum(o * o, axis=-1, keepdims=True))
    o_ref[...] = o / jnp.maximum(nrm, 1e-12)


def _full(shape):
    return pl.BlockSpec(shape, lambda i: (0, 0))


def _rows(width):
    return pl.BlockSpec((ROW_BLK, width), lambda i: (i, 0))


_GRID = N // ROW_BLK

_pre_call = pl.pallas_call(
    _pre_body,
    grid=(_GRID,),
    in_specs=[_rows(D_IN), _full((D_IN, 2 * D_HID)), _full((1, 2 * D_HID)),
              _full((D_HID, D_HID))],
    out_specs=[_rows(D_HID), _rows(D_HID), _rows(D_HID)],
    out_shape=[jax.ShapeDtypeStruct((N, D_HID), jnp.float32)] * 3,
)

_upd_call = pl.pallas_call(
    _upd_body,
    grid=(_GRID,),
    in_specs=[_rows(D_HID), _rows(D_HID), _rows(D_HID),
              _full((D_HID, D_HID)), _full((D_HID, D_HID)),
              _full((1, D_HID)), _full((D_HID, D_HID))],
    out_specs=[_rows(D_HID), _rows(D_HID)],
    out_shape=[jax.ShapeDtypeStruct((N, D_HID), jnp.float32)] * 2,
)

_out_call = pl.pallas_call(
    _out_body,
    grid=(_GRID,),
    in_specs=[_rows(D_HID), _full((D_HID, D_OUT)), _full((1, D_OUT))],
    out_specs=_rows(D_OUT),
    out_shape=jax.ShapeDtypeStruct((N, D_OUT), jnp.float32),
)


def _part_body(src_hbm, dst_hbm, ef_hbm,
               srcl_hbm, dstl_hbm, efl_hbm, cnt_hbm,
               sbuf, dbuf, ebuf, sst, dst_st, est, cntv):
    wid = lax.axis_index("s") * 2 + lax.axis_index("c")
    lo = wid * NPS

    def _shift(st_ref):
        def _mv(i, _):
            st_ref[pl.ds(i * 16, 16)] = st_ref[pl.ds(FLUSH + i * 16, 16)]
            return 0
        lax.fori_loop(0, FLUSH // 16, _mv, 0)

    def _chunk(c, carry):
        vptr, optr = carry
        off = c * SCHUNK
        pltpu.sync_copy(src_hbm.at[pl.ds(off, SCHUNK)], sbuf)
        pltpu.sync_copy(dst_hbm.at[pl.ds(off, SCHUNK)], dbuf)
        pltpu.sync_copy(ef_hbm.at[pl.ds(off, SCHUNK)], ebuf)

        def _grp(gi, vp):
            sl = pl.ds(gi * 16, 16)
            d16 = dbuf[sl]
            m = (d16 >= lo) & (d16 < lo + NPS)
            plsc.store_compressed(sst.at[pl.ds(vp, 16)], sbuf[sl], mask=m)
            plsc.store_compressed(dst_st.at[pl.ds(vp, 16)], d16 - lo, mask=m)
            plsc.store_compressed(est.at[pl.ds(vp, 16)], ebuf[sl], mask=m)
            pc = plsc.all_reduce_population_count(m)[0]
            return vp + pc
        vptr = lax.fori_loop(0, SCHUNK // 16, _grp, vptr)

        flushed = vptr >= FLUSH

        obase = pl.multiple_of(wid * CAP + optr, FLUSH)

        @pl.when(flushed)
        def _():
            pltpu.sync_copy(sst.at[pl.ds(0, FLUSH)],
                            srcl_hbm.at[pl.ds(obase, FLUSH)])
            pltpu.sync_copy(dst_st.at[pl.ds(0, FLUSH)],
                            dstl_hbm.at[pl.ds(obase, FLUSH)])
            pltpu.sync_copy(est.at[pl.ds(0, FLUSH)],
                            efl_hbm.at[pl.ds(obase, FLUSH)])
            _shift(sst)
            _shift(dst_st)
            _shift(est)

        vptr = jnp.where(flushed, vptr - FLUSH, vptr)
        optr = jnp.where(flushed, optr + FLUSH, optr)
        return vptr, optr

    vptr, optr = lax.fori_loop(0, NCH, _chunk,
                               (jnp.int32(0), jnp.int32(0)))

    for k in range(CHUNK // 16):
        at = pl.ds(vptr + k * 16, 16)
        sst[at] = jnp.zeros((16,), jnp.int32)
        dst_st[at] = jnp.full((16,), NPS, jnp.int32)
        est[at] = jnp.zeros((16,), jnp.float32)

    fbase = pl.multiple_of(wid * CAP + optr, FLUSH)
    pltpu.sync_copy(sst.at[pl.ds(0, FLUSH)],
                    srcl_hbm.at[pl.ds(fbase, FLUSH)])
    pltpu.sync_copy(dst_st.at[pl.ds(0, FLUSH)],
                    dstl_hbm.at[pl.ds(fbase, FLUSH)])
    pltpu.sync_copy(est.at[pl.ds(0, FLUSH)],
                    efl_hbm.at[pl.ds(fbase, FLUSH)])

    total = optr + ((vptr + CHUNK - 1) // CHUNK) * CHUNK
    cntv[...] = jnp.full((16,), total, jnp.int32).astype(jnp.float32)
    pltpu.sync_copy(cntv, cnt_hbm.at[pl.ds(wid * 16, 16)])


_part_call = pl.kernel(
    _part_body,
    out_type=[
        jax.ShapeDtypeStruct((NW * CAP,), jnp.int32),
        jax.ShapeDtypeStruct((NW * CAP,), jnp.int32),
        jax.ShapeDtypeStruct((NW * CAP,), jnp.float32),
        jax.ShapeDtypeStruct((NW * 16,), jnp.float32),
    ],
    mesh=plsc.VectorSubcoreMesh(core_axis_name="c", subcore_axis_name="s",
                                num_cores=2, num_subcores=16),
    compiler_params=pltpu.CompilerParams(needs_layout_passes=False),
    scratch_types=[
        pltpu.VMEM((SCHUNK,), jnp.int32),
        pltpu.VMEM((SCHUNK,), jnp.int32),
        pltpu.VMEM((SCHUNK,), jnp.float32),
        pltpu.VMEM((STAGE,), jnp.int32),
        pltpu.VMEM((STAGE,), jnp.int32),
        pltpu.VMEM((STAGE,), jnp.float32),
        pltpu.VMEM((16,), jnp.float32),
    ],
)


def _sort_body(srcl_hbm, dstl_hbm, efl_hbm, cnt_hbm,
               srcs_hbm, dsts_hbm, efs_hbm,
               sbuf, dbuf, ebuf, hist_a, hist_b, wptr,
               s_out, d_out, e_out, cntv):
    wid = lax.axis_index("s") * 2 + lax.axis_index("c")
    base = wid * CAP
    iota = lax.iota(jnp.int32, 16)

    pltpu.sync_copy(cnt_hbm.at[pl.ds(wid * 16, 16)], cntv)
    cnt = cntv[...][0].astype(jnp.int32)
    nscan = (cnt + HCH - 1) // HCH

    def _zh(i, _):
        z = jnp.zeros((16,), jnp.int32)
        hist_a[pl.ds(i * 16, 16)] = z
        hist_b[pl.ds(i * 16, 16)] = z
        return 0
    lax.fori_loop(0, (NPS + 16) // 16, _zh, 0)

    def _hchunk(c, _):
        off = pl.multiple_of(base + c * HCH, 8)
        gbase = c * HCH
        pltpu.sync_copy(dstl_hbm.at[pl.ds(off, HCH)], dbuf)

        def _hgrp(g, _):
            d16 = dbuf[pl.ds(g * 16, 16)]
            valid = (gbase + g * 16 + iota) < cnt
            d16c = jnp.where(valid, jnp.clip(d16, 0, NPS), NPS)
            for j2 in range(16):
                bank = hist_a if j2 % 2 == 0 else hist_b
                dsp = jnp.full((16,), d16c[j2], jnp.int32)
                cv = plsc.load_gather(bank, [dsp])
                plsc.store_scatter(bank, [dsp], cv + 1)
            return 0
        lax.fori_loop(0, HCH // 16, _hgrp, 0)
        return 0
    lax.fori_loop(0, nscan, _hchunk, 0)

    def _prefix():
        carry = jnp.int32(0)
        for g in range((NPS + 16) // 16):
            h16 = hist_a[pl.ds(g * 16, 16)] + hist_b[pl.ds(g * 16, 16)]
            vals = jnp.zeros((16,), jnp.int32)
            for j2 in range(16):
                vals = jnp.where(iota == j2, carry, vals)
                carry = carry + h16[j2]
            wptr[pl.ds(g * 16, 16)] = vals
        return carry
    _prefix()

    nrounds = (cnt + WIN - 1) // WIN

    def _round(r, _):
        lo_pos = r * WIN

        def _cpy(i, _):
            return 0
        _prefix()

        def _schunk(c, _):
            off = pl.multiple_of(base + c * HCH, 8)
            gbase = c * HCH
            pltpu.sync_copy(srcl_hbm.at[pl.ds(off, HCH)], sbuf)
            pltpu.sync_copy(dstl_hbm.at[pl.ds(off, HCH)], dbuf)
            pltpu.sync_copy(efl_hbm.at[pl.ds(off, HCH)], ebuf)

            def _sgrp(g, _):
                sl = pl.ds(g * 16, 16)
                d16 = dbuf[sl]
                s16 = sbuf[sl]
                e16 = ebuf[sl]
                valid = (gbase + g * 16 + iota) < cnt
                d16c = jnp.where(valid, jnp.clip(d16, 0, NPS), NPS)
                lane0 = iota == 0
                for j2 in range(16):
                    dsp = jnp.full((16,), d16c[j2], jnp.int32)
                    pos = plsc.load_gather(wptr, [dsp])
                    plsc.store_scatter(wptr, [dsp], pos + 1)
                    inw = (pos >= lo_pos) & (pos < lo_pos + WIN)
                    m = inw & lane0
                    pl_loc = pos - lo_pos
                    plsc.store_scatter(s_out, [pl_loc],
                                       jnp.full((16,), s16[j2], jnp.int32),
                                       mask=m)
                    plsc.store_scatter(d_out, [pl_loc],
                                       jnp.full((16,), d16c[j2], jnp.int32),
                                       mask=m)
                    plsc.store_scatter(e_out, [pl_loc],
                                       jnp.full((16,), e16[j2], jnp.float32),
                                       mask=m)
                return 0
            lax.fori_loop(0, HCH // 16, _sgrp, 0)
            return 0
        lax.fori_loop(0, nscan, _schunk, 0)

        wbase = pl.multiple_of(base + r * WIN, 8)
        pltpu.sync_copy(s_out, srcs_hbm.at[pl.ds(wbase, WIN)])
        pltpu.sync_copy(d_out, dsts_hbm.at[pl.ds(wbase, WIN)])
        pltpu.sync_copy(e_out, efs_hbm.at[pl.ds(wbase, WIN)])
        return 0
    lax.fori_loop(0, nrounds, _round, 0)


_sort_call = pl.kernel(
    _sort_body,
    out_type=[
        jax.ShapeDtypeStruct((NW * CAP,), jnp.int32),
        jax.ShapeDtypeStruct((NW * CAP,), jnp.int32),
        jax.ShapeDtypeStruct((NW * CAP,), jnp.float32),
    ],
    mesh=plsc.VectorSubcoreMesh(core_axis_name="c", subcore_axis_name="s",
                                num_cores=2, num_subcores=16),
    compiler_params=pltpu.CompilerParams(needs_layout_passes=False),
    scratch_types=[
        pltpu.VMEM((HCH,), jnp.int32),
        pltpu.VMEM((HCH,), jnp.int32),
        pltpu.VMEM((HCH,), jnp.float32),
        pltpu.VMEM((NPS + 16,), jnp.int32),
        pltpu.VMEM((NPS + 16,), jnp.int32),
        pltpu.VMEM((NPS + 16,), jnp.int32),
        pltpu.VMEM((WIN,), jnp.int32),
        pltpu.VMEM((WIN,), jnp.int32),
        pltpu.VMEM((WIN,), jnp.float32),
        pltpu.VMEM((16,), jnp.float32),
    ],
)


def _edge_body(q_hbm, src_hbm, dloc_hbm, ef_hbm, cnt_hbm, wvec_hbm,
               mhat_hbm,
               wv_v, cnt_v, sidx_v, dloc_v, ef_v, rows_v, acc_v, sem):
    wid = lax.axis_index("s") * 2 + lax.axis_index("c")

    pltpu.sync_copy(wvec_hbm, wv_v)
    pltpu.sync_copy(cnt_hbm.at[pl.ds(wid * 16, 16)], cnt_v)

    def _zero(i, _):
        for k in range(8):
            acc_v[i, pl.ds(k * 16, 16)] = jnp.zeros((16,), jnp.float32)
        return 0
    lax.fori_loop(0, NPS + 1, _zero, 0)

    nchunks = cnt_v[...][0].astype(jnp.int32) // CHUNK
    iota = lax.iota(jnp.int32, 16)
    wregs = [wv_v[pl.ds(k * 16, 16)] for k in range(8)]

    def _issue(c, sl):
        off = pl.multiple_of(wid * CAP + c * CHUNK, CHUNK)
        pltpu.sync_copy(src_hbm.at[pl.ds(off, CHUNK)], sidx_v.at[sl])
        pltpu.async_copy(q_hbm.at[sidx_v.at[sl]], rows_v.at[sl], sem.at[sl])
        pltpu.sync_copy(dloc_hbm.at[pl.ds(off, CHUNK)], dloc_v.at[sl])
        pltpu.sync_copy(ef_hbm.at[pl.ds(off, CHUNK)], ef_v.at[sl])

    for pr in range(3):
        @pl.when(pr < nchunks)
        def _(pr=pr):
            _issue(pr, pr)

    def _chunk(c, carry):
        sl = c % 4

        @pl.when(c + 3 < nchunks)
        def _():
            _issue(c + 3, (c + 3) % 4)

        pltpu.make_async_copy(q_hbm.at[sidx_v.at[sl]], rows_v.at[sl],
                              sem.at[sl]).wait()

        def _grp(gidx, carry):
            prev = carry[0]
            accs = carry[1:]
            d16 = dloc_v[sl, pl.ds(gidx * 16, 16)]
            e16 = ef_v[sl, pl.ds(gidx * 16, 16)]
            for j2 in range(16):
                j = gidx * 16 + j2
                dvec = jnp.full((16,), d16[j2], jnp.int32)
                evec = jnp.full((16,), e16[j2], jnp.float32)
                is_new = dvec != prev
                new_accs = []
                for k in range(8):
                    rk = rows_v[sl, j, pl.ds(k * 16, 16)]
                    ik = iota + (k * 16)
                    vk = rk + evec * wregs[k]
                    plsc.store_scatter(acc_v, [prev, ik], accs[k],
                                       mask=is_new)
                    ak = jnp.where(is_new, vk, jnp.maximum(accs[k], vk))
                    new_accs.append(ak)
                prev = dvec
                accs = new_accs
            return (prev,) + tuple(accs)
        carry = lax.fori_loop(0, CHUNK // 16, _grp, carry)
        return carry

    nps_sp = jnp.full((16,), NPS, jnp.int32)
    zero_v = jnp.zeros((16,), jnp.float32)
    carry0 = (nps_sp,) + (zero_v,) * 8
    carry = lax.fori_loop(0, nchunks, _chunk, carry0)

    prev = carry[0]
    for k in range(8):
        ik = iota + (k * 16)
        plsc.store_scatter(acc_v, [prev, ik], carry[1 + k])

    def _clamp(i, _):
        for k in range(8):
            sle = pl.ds(k * 16, 16)
            acc_v[i, sle] = jnp.maximum(acc_v[i, sle], 0.0)
        return 0
    lax.fori_loop(0, NPS, _clamp, 0)
    pltpu.sync_copy(acc_v.at[pl.ds(0, NPS)], mhat_hbm.at[pl.ds(wid * NPS, NPS)])


_edge_call = pl.kernel(
    _edge_body,
    out_type=jax.ShapeDtypeStruct((NPAD, D_HID), jnp.float32),
    mesh=plsc.VectorSubcoreMesh(core_axis_name="c", subcore_axis_name="s",
                                num_cores=2, num_subcores=16),
    compiler_params=pltpu.CompilerParams(needs_layout_passes=False),
    scratch_types=[
        pltpu.VMEM((D_HID,), jnp.float32),
        pltpu.VMEM((16,), jnp.float32),
        pltpu.VMEM((4, CHUNK), jnp.int32),
        pltpu.VMEM((4, CHUNK), jnp.int32),
        pltpu.VMEM((4, CHUNK), jnp.float32),
        pltpu.VMEM((4, CHUNK, D_HID), jnp.float32),
        pltpu.VMEM((NPS + 1, D_HID), jnp.float32),
        pltpu.SemaphoreType.DMA((4,)),
    ],
)


@jax.jit
def _run(state, edge_index, node_feature, edge_feature,
         W_in, b_in, W_msg, b_msg, W_upd, b_upd, W_out, b_out):
    glob = jnp.broadcast_to(state[0, :5][None, :], (N, 5))
    loc1 = state[0, 5:5 + N][:, None]
    loc2 = state[0, 5 + N:5 + 2 * N][:, None]
    ni = jnp.concatenate([glob, loc1, loc2, node_feature], axis=-1)

    W_h = W_msg[:D_HID]
    w_e = W_msg[D_HID]
    W_s = W_msg[D_HID + 1:]
    Wcat = jnp.concatenate([W_in, W_s], axis=1)
    bcat = jnp.concatenate([b_in, b_msg])[None, :]
    Wu1 = W_upd[:D_HID]
    Wu2 = W_upd[D_HID:]

    src_l, dst_l, ef_l, counts = _part_call(
        edge_index[0], edge_index[1], edge_feature)
    src_l, dst_l, ef_l = _sort_call(src_l, dst_l, ef_l, counts)

    h, g, q = _pre_call(ni, Wcat, bcat, W_h)
    for _ in range(N_ITERS):
        mhat = _edge_call(q, src_l, dst_l, ef_l, counts, w_e)[:N]
        h, q = _upd_call(mhat, h, g, Wu1, Wu2, b_upd[None, :], W_h)
    out = _out_call(h, W_out, b_out[None, :])
    return out[:, None, :]


def kernel(state, edge_index, node_feature, edge_feature,
           W_in, b_in, W_msg, b_msg, W_upd, b_upd, W_out, b_out):
    if state.ndim == 1:
        state = state[None, :]
    return _run(state, edge_index, node_feature, edge_feature,
                W_in, b_in, W_msg, b_msg, W_upd, b_upd, W_out, b_out)
